# Initial kernel scaffold; baseline (speedup 1.0000x reference)
import jax
import jax.numpy as jnp
from jax.experimental import pallas as pl


def kernel(x_operator, x_table, x_column, x_predicate, ei_scannedby, ei_filters, ei_outputby, ei_connects, ei_calledby, ei_sl_table, ei_sl_column, batch_operator, W_op, b_op, W_tab, b_tab, W_col, b_col, W_pred, b_pred, Wrel1, brel1, Wroot1, Wrel2, brel2, Wroot2, W_out, b_out):
    raise NotImplementedError("write your pallas kernel here")


# speedup vs baseline: 1.5025x; 1.5025x over previous
import jax
import jax.numpy as jnp
from jax import lax
from jax.experimental import pallas as pl
from jax.experimental.pallas import tpu as pltpu
from jax.experimental.pallas import tpu_sc as plsc

F32 = jnp.float32
I32 = jnp.int32

H = 128
G = 64
NOP, NTAB, NCOL, NPRED = 100000, 20000, 100000, 50000

_JW = 128
_JROWS = 8
_CHUNK = _JW * _JROWS
_NSUB = 16

NOP_P = 100096
NTAB_P = 20096
NPRED_P = 50048
NCOL_P = 100096
_NZR = NOP_P // _NSUB

_mesh = plsc.VectorSubcoreMesh(
    core_axis_name="c", subcore_axis_name="s", num_cores=2, num_subcores=16)


def _edge_chunks(es, ed, raw, acc, idx_s, idx_d, rows, sid):
    nrows = es.shape[0]
    nchunks = nrows // _JROWS
    nk = (nchunks + _NSUB - 1) // _NSUB

    def kbody(k, carry):
        c = k * _NSUB + sid

        @pl.when(c < nchunks)
        def _():
            pltpu.sync_copy(es.at[pl.ds(c * _JROWS, _JROWS), :], idx_s)
            pltpu.sync_copy(ed.at[pl.ds(c * _JROWS, _JROWS), :], idx_d)

            def jbody(j, carry2):
                pltpu.sync_copy(raw.at[idx_s.at[j]], rows)
                pltpu.sync_copy(rows, acc.at[idx_d.at[j]], add=True)
                return carry2

            lax.fori_loop(0, _JROWS, jbody, 0)
        return carry

    lax.fori_loop(0, nk, kbody, 0)


def _sc_agg1_body(raw_op, raw_tab, raw_col, raw_pred, zeros,
                  es0, ed0, es1, ed1, es2, ed2, es3, ed3, es4, ed4,
                  es5, ed5, es6, ed6,
                  agg0, agg1, agg2, agg3, agg4, agg5, agg6,
                  acc, idx_s, idx_d, rows):
    cid = lax.axis_index("c")
    sid = lax.axis_index("s")
    rels = [
        (es0, ed0, agg0, raw_tab, NOP_P, 0),
        (es2, ed2, agg2, raw_col, NOP_P, 0),
        (es3, ed3, agg3, raw_col, NPRED_P, 0),
        (es1, ed1, agg1, raw_pred, NOP_P, 1),
        (es4, ed4, agg4, raw_op, NOP_P, 1),
        (es5, ed5, agg5, raw_tab, NTAB_P, 1),
        (es6, ed6, agg6, raw_col, NCOL_P, 1),
    ]
    for es, ed, agg, raw, ndst, core in rels:
        @pl.when(cid == core)
        def _(es=es, ed=ed, agg=agg, raw=raw, ndst=ndst):
            nr = ndst // _NSUB
            pltpu.sync_copy(zeros.at[pl.ds(0, nr)],
                            acc.at[pl.ds(sid * nr, nr)])
            plsc.subcore_barrier()
            _edge_chunks(es, ed, raw, acc, idx_s, idx_d, rows, sid)
            plsc.subcore_barrier()
            pltpu.sync_copy(acc.at[pl.ds(sid * nr, nr)],
                            agg.at[pl.ds(sid * nr, nr)])
            plsc.subcore_barrier()


def _sc_agg1(raw16s, zeros, edges):
    ndsts = [NOP_P, NOP_P, NOP_P, NPRED_P, NOP_P, NTAB_P, NCOL_P]
    out_type = [jax.ShapeDtypeStruct((n, 16), F32) for n in ndsts]
    flat_edges = [a for pair in edges for a in pair]
    f = pl.kernel(
        _sc_agg1_body,
        out_type=out_type,
        mesh=_mesh,
        scratch_types=[
            pltpu.VMEM_SHARED((NOP_P, 16), F32),
            pltpu.VMEM((_JROWS, _JW), I32),
            pltpu.VMEM((_JROWS, _JW), I32),
            pltpu.VMEM((_JW, 16), F32),
        ],
        compiler_params=pltpu.CompilerParams(use_tc_tiling_on_sc=False),
    )
    return f(*raw16s, zeros, *flat_edges)


def _sc_agg2_body(*refs):
    y2 = {}
    k = 0
    for t in ("tab", "pred", "col", "op"):
        y2[t] = refs[k:k + 8]
        k += 8
    es0, ed0, es1, ed1, es2, ed2, es4, ed4, zeros = refs[k:k + 9]
    k += 9
    outs = refs[k:k + 8]
    k += 8
    acc, idx_s, idx_d, rows = refs[k:k + 4]

    cid = lax.axis_index("c")
    sid = lax.axis_index("s")
    nr = NOP_P // _NSUB
    rel_edges = [(es0, ed0, "tab"), (es1, ed1, "pred"),
                 (es2, ed2, "col"), (es4, ed4, "op")]
    for s in range(8):
        @pl.when(cid == s // 4)
        def _(s=s):
            pltpu.sync_copy(zeros.at[pl.ds(0, nr)],
                            acc.at[pl.ds(sid * nr, nr)])
            plsc.subcore_barrier()
            for es, ed, t in rel_edges:
                _edge_chunks(es, ed, y2[t][s], acc, idx_s, idx_d, rows, sid)
            plsc.subcore_barrier()
            pltpu.sync_copy(acc.at[pl.ds(sid * nr, nr)],
                            outs[s].at[pl.ds(sid * nr, nr)])
            plsc.subcore_barrier()


def _sc_agg2(y2_slices, edges, zeros):
    out_type = [jax.ShapeDtypeStruct((NOP_P, 16), F32) for _ in range(8)]
    args = []
    for t in ("tab", "pred", "col", "op"):
        args.extend(y2_slices[t])
    for es, ed in edges:
        args.extend((es, ed))
    args.append(zeros)
    f = pl.kernel(
        _sc_agg2_body,
        out_type=out_type,
        mesh=_mesh,
        scratch_types=[
            pltpu.VMEM_SHARED((NOP_P, 16), F32),
            pltpu.VMEM((_JROWS, _JW), I32),
            pltpu.VMEM((_JROWS, _JW), I32),
            pltpu.VMEM((_JW, 16), F32),
        ],
        compiler_params=pltpu.CompilerParams(use_tc_tiling_on_sc=False),
    )
    return f(*args)


_BLK = 2000


def _combine1(aggs, raw16, a1s, r1, cb1, wrel2, n, want_h1):
    nb = n // _BLK
    na = len(aggs)

    def body(*refs):
        ins = refs[:na + 1]
        ws = refs[na + 1:2 * na + 1]
        r1_ref, cb1_ref, w2_ref = refs[2 * na + 1:2 * na + 4]
        outs = refs[2 * na + 4:]
        h = jnp.dot(ins[na][...], r1_ref[...],
                    preferred_element_type=F32) + cb1_ref[...]
        for a in range(na):
            h += jnp.dot(ins[a][...], ws[a][...], preferred_element_type=F32)
        h = jnp.maximum(h, 0.0)
        y2 = jnp.dot(h, w2_ref[...], preferred_element_type=F32)
        for s in range(8):
            outs[s][...] = y2[:, s * 16:(s + 1) * 16]
        if want_h1:
            outs[8][...] = h

    agg_spec = pl.BlockSpec((_BLK, 16), lambda i: (i, 0))
    w_spec = pl.BlockSpec((16, H), lambda i: (0, 0))
    out_shapes = [jax.ShapeDtypeStruct((n, 16), F32) for _ in range(8)]
    out_specs = [agg_spec] * 8
    if want_h1:
        out_shapes.append(jax.ShapeDtypeStruct((n, H), F32))
        out_specs.append(pl.BlockSpec((_BLK, H), lambda i: (i, 0)))
    res = pl.pallas_call(
        body,
        grid=(nb,),
        in_specs=([agg_spec] * (na + 1) + [w_spec] * (na + 1)
                  + [pl.BlockSpec((1, H), lambda i: (0, 0)),
                     pl.BlockSpec((H, H), lambda i: (0, 0))]),
        out_specs=out_specs,
        out_shape=out_shapes,
    )(*aggs, raw16, *a1s, r1, cb1, wrel2)
    slices = list(res[:8])
    return (slices, res[8]) if want_h1 else (slices, None)


def _head(acc2_slices, h1_op, batch3d, wroot2s, bias2, w_out):
    nb = NOP // _BLK

    def body(*refs):
        accs = refs[:8]
        h1_ref, b_ref, wr_ref, bias_ref, wout_ref = refs[8:13]
        sums_ref, cnts_ref = refs[13:15]
        i = pl.program_id(0)
        acc = jnp.concatenate([accs[s][...] for s in range(8)], axis=1)
        h2 = jnp.maximum(
            acc + jnp.dot(h1_ref[...], wr_ref[...],
                          preferred_element_type=F32) + bias_ref[...], 0.0)
        z = jnp.dot(h2, wout_ref[...], preferred_element_type=F32)
        b = b_ref[0]
        oh = (lax.broadcasted_iota(I32, (G, _BLK), 0) == b).astype(F32)
        ps = lax.dot_general(oh, z, (((1,), (0,)), ((), ())),
                             preferred_element_type=F32)
        pc = jnp.sum(oh, axis=1, keepdims=True)

        @pl.when(i == 0)
        def _():
            sums_ref[...] = ps
            cnts_ref[...] = pc

        @pl.when(i > 0)
        def _():
            sums_ref[...] += ps
            cnts_ref[...] += pc

    slice_spec = pl.BlockSpec((_BLK, 16), lambda i: (i, 0))
    out_spec = pl.BlockSpec((G, 1), lambda i: (0, 0))
    sums, cnts = pl.pallas_call(
        body,
        grid=(nb,),
        in_specs=([slice_spec] * 8
                  + [pl.BlockSpec((_BLK, H), lambda i: (i, 0)),
                     pl.BlockSpec((1, 1, _BLK), lambda i: (i, 0, 0)),
                     pl.BlockSpec((H, H), lambda i: (0, 0)),
                     pl.BlockSpec((1, H), lambda i: (0, 0)),
                     pl.BlockSpec((H, 1), lambda i: (0, 0))]),
        out_specs=[out_spec, out_spec],
        out_shape=[jax.ShapeDtypeStruct((G, 1), F32),
                   jax.ShapeDtypeStruct((G, 1), F32)],
    )(*acc2_slices, h1_op, batch3d, wroot2s, bias2, w_out)
    return sums, cnts


def _pad16(w, b):
    p = jnp.concatenate([w, b[None, :]], axis=0)
    return jnp.pad(p, ((0, 16 - p.shape[0]), (0, 0)))


def _raw16(x):
    n, d = x.shape
    return jnp.concatenate(
        [x, jnp.ones((n, 1), F32), jnp.zeros((n, 15 - d), F32)], axis=1)


def kernel(x_operator, x_table, x_column, x_predicate, ei_scannedby,
           ei_filters, ei_outputby, ei_connects, ei_calledby, ei_sl_table,
           ei_sl_column, batch_operator, W_op, b_op, W_tab, b_tab, W_col,
           b_col, W_pred, b_pred, Wrel1, brel1, Wroot1, Wrel2, brel2,
           Wroot2, W_out, b_out):
    raw_op = _raw16(x_operator)
    raw_tab = _raw16(x_table)
    raw_col = _raw16(x_column)
    raw_pred = _raw16(x_predicate)
    zeros = jnp.zeros((_NZR, 16), F32)

    ndst_real = [NOP, NOP, NOP, NPRED, NOP, NTAB, NCOL]
    eis = [ei_scannedby, ei_filters, ei_outputby, ei_connects, ei_calledby,
           ei_sl_table, ei_sl_column]
    edges = []
    for e, nd in zip(eis, ndst_real):
        ne = e.shape[1]
        npad = (-ne) % _CHUNK
        es = jnp.concatenate([e[0], jnp.zeros((npad,), I32)])
        ed = jnp.concatenate([e[1], jnp.full((npad,), nd, I32)])
        edges.append((es.reshape(-1, _JW), ed.reshape(-1, _JW)))

    p16 = {"op": _pad16(W_op, b_op), "tab": _pad16(W_tab, b_tab),
           "col": _pad16(W_col, b_col), "pred": _pad16(W_pred, b_pred)}
    srcs = ["tab", "pred", "col", "col", "op", "tab", "col"]
    a1 = [p16[srcs[i]] @ Wrel1[i] for i in range(7)]
    rel_of_dst = {"op": [0, 1, 2, 4], "pred": [3], "tab": [5], "col": [6]}
    r1 = {}
    cb1 = {}
    for t, rl in rel_of_dst.items():
        wroot_sum = sum(Wroot1[i] for i in rl)
        r1[t] = p16[t] @ wroot_sum
        cb1[t] = sum(brel1[i] for i in rl)[None, :]

    aggs = _sc_agg1([raw_op, raw_tab, raw_col, raw_pred], zeros, edges)

    y2 = {}
    y2["op"], h1_op = _combine1(
        [aggs[0], aggs[1], aggs[2], aggs[4]], raw_op,
        [a1[0], a1[1], a1[2], a1[4]], r1["op"], cb1["op"], Wrel2[4],
        NOP, True)
    y2["tab"], _ = _combine1([aggs[5]], raw_tab, [a1[5]], r1["tab"],
                             cb1["tab"], Wrel2[0], NTAB, False)
    y2["pred"], _ = _combine1([aggs[3]], raw_pred, [a1[3]], r1["pred"],
                              cb1["pred"], Wrel2[1], NPRED, False)
    y2["col"], _ = _combine1([aggs[6]], raw_col, [a1[6]], r1["col"],
                             cb1["col"], Wrel2[2], NCOL, False)

    acc2 = _sc_agg2(y2, [edges[0], edges[1], edges[2], edges[4]], zeros)

    wroot2s = Wroot2[0] + Wroot2[1] + Wroot2[2] + Wroot2[4]
    bias2 = (brel2[0] + brel2[1] + brel2[2] + brel2[4])[None, :]
    batch3d = batch_operator.reshape(NOP // _BLK, 1, _BLK)
    sums, cnts = _head(acc2, h1_op, batch3d, wroot2s, bias2, W_out)
    return sums[:, 0] / jnp.maximum(cnts[:, 0], 1.0) + b_out


# speedup vs baseline: 2.3926x; 1.5924x over previous
import jax
import jax.numpy as jnp
from jax import lax
from jax.experimental import pallas as pl
from jax.experimental.pallas import tpu as pltpu
from jax.experimental.pallas import tpu_sc as plsc

F32 = jnp.float32
I32 = jnp.int32

H = 128
G = 64
NOP, NTAB, NCOL, NPRED = 100000, 20000, 100000, 50000

_JW = 128
_JROWS = 8
_CHUNK = _JW * _JROWS
_NSUB = 16

NOP_P = 100096
NTAB_P = 20096
NPRED_P = 50048
NCOL_P = 100096
_NZR = NOP_P // _NSUB

_mesh = plsc.VectorSubcoreMesh(
    core_axis_name="c", subcore_axis_name="s", num_cores=2, num_subcores=16)


def _edge_chunks(es, ed, raw, acc, idx_s, idx_d, rows, sid, col=None):
    nrows = es.shape[0]
    nchunks = nrows // _JROWS
    nk = (nchunks + _NSUB - 1) // _NSUB

    def kbody(k, carry):
        c = k * _NSUB + sid

        @pl.when(c < nchunks)
        def _():
            pltpu.sync_copy(es.at[pl.ds(c * _JROWS, _JROWS), :], idx_s)
            pltpu.sync_copy(ed.at[pl.ds(c * _JROWS, _JROWS), :], idx_d)
            if col is not None:
                for j in range(_JROWS):
                    for l in range(_JW // 16):
                        v = idx_s[j, pl.ds(l * 16, 16)]
                        idx_s[j, pl.ds(l * 16, 16)] = v * 8 + col

            def jbody(j, carry2):
                pltpu.sync_copy(raw.at[idx_s.at[j]], rows)
                pltpu.sync_copy(rows, acc.at[idx_d.at[j]], add=True)
                return carry2

            lax.fori_loop(0, _JROWS, jbody, 0)
        return carry

    lax.fori_loop(0, nk, kbody, 0)


def _sc_agg1_body(raw_op, raw_tab, raw_col, raw_pred, zeros,
                  es0, ed0, es1, ed1, es2, ed2, es3, ed3, es4, ed4,
                  es5, ed5, es6, ed6,
                  agg0, agg1, agg2, agg3, agg4, agg5, agg6,
                  acc, idx_s, idx_d, rows):
    cid = lax.axis_index("c")
    sid = lax.axis_index("s")
    rels = [
        (es0, ed0, agg0, raw_tab, NOP_P, 0),
        (es2, ed2, agg2, raw_col, NOP_P, 0),
        (es3, ed3, agg3, raw_col, NPRED_P, 0),
        (es1, ed1, agg1, raw_pred, NOP_P, 1),
        (es4, ed4, agg4, raw_op, NOP_P, 1),
        (es5, ed5, agg5, raw_tab, NTAB_P, 1),
        (es6, ed6, agg6, raw_col, NCOL_P, 1),
    ]
    for es, ed, agg, raw, ndst, core in rels:
        @pl.when(cid == core)
        def _(es=es, ed=ed, agg=agg, raw=raw, ndst=ndst):
            nr = ndst // _NSUB
            pltpu.sync_copy(zeros.at[pl.ds(0, nr)],
                            acc.at[pl.ds(sid * nr, nr)])
            plsc.subcore_barrier()
            _edge_chunks(es, ed, raw, acc, idx_s, idx_d, rows, sid)
            plsc.subcore_barrier()
            pltpu.sync_copy(acc.at[pl.ds(sid * nr, nr)],
                            agg.at[pl.ds(sid * nr, nr)])
            plsc.subcore_barrier()


def _sc_agg1(raw16s, zeros, edges):
    ndsts = [NOP_P, NOP_P, NOP_P, NPRED_P, NOP_P, NTAB_P, NCOL_P]
    out_type = [jax.ShapeDtypeStruct((n, 16), F32) for n in ndsts]
    flat_edges = [a for pair in edges for a in pair]
    f = pl.kernel(
        _sc_agg1_body,
        out_type=out_type,
        mesh=_mesh,
        scratch_types=[
            pltpu.VMEM_SHARED((NOP_P, 16), F32),
            pltpu.VMEM((_JROWS, _JW), I32),
            pltpu.VMEM((_JROWS, _JW), I32),
            pltpu.VMEM((_JW, 16), F32),
        ],
        compiler_params=pltpu.CompilerParams(use_tc_tiling_on_sc=False),
    )
    return f(*raw16s, zeros, *flat_edges)


def _sc_agg2_body(y2_tab, y2_pred, y2_col, y2_op,
                  es0, ed0, es1, ed1, es2, ed2, es4, ed4, zeros, out,
                  acc, idx_s, idx_d, rows):
    cid = lax.axis_index("c")
    sid = lax.axis_index("s")
    nr = NOP_P // _NSUB
    rel_edges = [(es0, ed0, y2_tab), (es1, ed1, y2_pred),
                 (es2, ed2, y2_col), (es4, ed4, y2_op)]
    for s in range(8):
        @pl.when(cid == s // 4)
        def _(s=s):
            pltpu.sync_copy(zeros.at[pl.ds(0, nr)],
                            acc.at[pl.ds(sid * nr, nr)])
            plsc.subcore_barrier()
            for es, ed, y2 in rel_edges:
                _edge_chunks(es, ed, y2, acc, idx_s, idx_d, rows, sid, col=s)
            plsc.subcore_barrier()
            pltpu.sync_copy(acc.at[pl.ds(sid * nr, nr)],
                            out.at[pl.ds(sid * nr, nr), pl.ds(s * 16, 16)])
            plsc.subcore_barrier()


def _sc_agg2(y2, edges, zeros):
    out_type = jax.ShapeDtypeStruct((NOP_P, H), F32)
    args = [y2[t].reshape(-1, 16) for t in ("tab", "pred", "col", "op")]
    for es, ed in edges:
        args.extend((es, ed))
    args.append(zeros)
    f = pl.kernel(
        _sc_agg2_body,
        out_type=out_type,
        mesh=_mesh,
        scratch_types=[
            pltpu.VMEM_SHARED((NOP_P, 16), F32),
            pltpu.VMEM((_JROWS, _JW), I32),
            pltpu.VMEM((_JROWS, _JW), I32),
            pltpu.VMEM((_JW, 16), F32),
        ],
        compiler_params=pltpu.CompilerParams(use_tc_tiling_on_sc=False),
    )
    return f(*args)


_BLK = 2000


def _combine1(aggs, raw16, a1s, r1, cb1, wrel2, n, want_h1):
    nb = n // _BLK
    na = len(aggs)

    def body(*refs):
        ins = refs[:na + 1]
        ws = refs[na + 1:2 * na + 1]
        r1_ref, cb1_ref, w2_ref = refs[2 * na + 1:2 * na + 4]
        outs = refs[2 * na + 4:]
        h = jnp.dot(ins[na][...], r1_ref[...],
                    preferred_element_type=F32) + cb1_ref[...]
        for a in range(na):
            h += jnp.dot(ins[a][...], ws[a][...], preferred_element_type=F32)
        h = jnp.maximum(h, 0.0)
        outs[0][...] = jnp.dot(h, w2_ref[...], preferred_element_type=F32)
        if want_h1:
            outs[1][...] = h

    agg_spec = pl.BlockSpec((_BLK, 16), lambda i: (i, 0))
    w_spec = pl.BlockSpec((16, H), lambda i: (0, 0))
    big_spec = pl.BlockSpec((_BLK, H), lambda i: (i, 0))
    out_shapes = [jax.ShapeDtypeStruct((n, H), F32)]
    out_specs = [big_spec]
    if want_h1:
        out_shapes.append(jax.ShapeDtypeStruct((n, H), F32))
        out_specs.append(big_spec)
    res = pl.pallas_call(
        body,
        grid=(nb,),
        in_specs=([agg_spec] * (na + 1) + [w_spec] * (na + 1)
                  + [pl.BlockSpec((1, H), lambda i: (0, 0)),
                     pl.BlockSpec((H, H), lambda i: (0, 0))]),
        out_specs=out_specs,
        out_shape=out_shapes,
    )(*aggs, raw16, *a1s, r1, cb1, wrel2)
    return (res[0], res[1]) if want_h1 else (res[0], None)


def _head(acc2, h1_op, batch3d, wroot2s, bias2, w_out):
    nb = NOP // _BLK

    def body(*refs):
        acc_ref, h1_ref, b_ref, wr_ref, bias_ref, wout_ref = refs[:6]
        sums_ref, cnts_ref = refs[6:8]
        i = pl.program_id(0)
        h2 = jnp.maximum(
            acc_ref[...] + jnp.dot(h1_ref[...], wr_ref[...],
                                   preferred_element_type=F32)
            + bias_ref[...], 0.0)
        z = jnp.dot(h2, wout_ref[...], preferred_element_type=F32)
        b = b_ref[0]
        oh = (lax.broadcasted_iota(I32, (G, _BLK), 0) == b).astype(F32)
        ps = lax.dot_general(oh, z, (((1,), (0,)), ((), ())),
                             preferred_element_type=F32)
        pc = jnp.sum(oh, axis=1, keepdims=True)

        @pl.when(i == 0)
        def _():
            sums_ref[...] = ps
            cnts_ref[...] = pc

        @pl.when(i > 0)
        def _():
            sums_ref[...] += ps
            cnts_ref[...] += pc

    out_spec = pl.BlockSpec((G, 1), lambda i: (0, 0))
    sums, cnts = pl.pallas_call(
        body,
        grid=(nb,),
        in_specs=[pl.BlockSpec((_BLK, H), lambda i: (i, 0)),
                  pl.BlockSpec((_BLK, H), lambda i: (i, 0)),
                  pl.BlockSpec((1, 1, _BLK), lambda i: (i, 0, 0)),
                  pl.BlockSpec((H, H), lambda i: (0, 0)),
                  pl.BlockSpec((1, H), lambda i: (0, 0)),
                  pl.BlockSpec((H, 1), lambda i: (0, 0))],
        out_specs=[out_spec, out_spec],
        out_shape=[jax.ShapeDtypeStruct((G, 1), F32),
                   jax.ShapeDtypeStruct((G, 1), F32)],
    )(acc2, h1_op, batch3d, wroot2s, bias2, w_out)
    return sums, cnts


def _pad16(w, b):
    p = jnp.concatenate([w, b[None, :]], axis=0)
    return jnp.pad(p, ((0, 16 - p.shape[0]), (0, 0)))


def _raw16(x):
    n, d = x.shape
    return jnp.concatenate(
        [x, jnp.ones((n, 1), F32), jnp.zeros((n, 15 - d), F32)], axis=1)


def kernel(x_operator, x_table, x_column, x_predicate, ei_scannedby,
           ei_filters, ei_outputby, ei_connects, ei_calledby, ei_sl_table,
           ei_sl_column, batch_operator, W_op, b_op, W_tab, b_tab, W_col,
           b_col, W_pred, b_pred, Wrel1, brel1, Wroot1, Wrel2, brel2,
           Wroot2, W_out, b_out):
    raw_op = _raw16(x_operator)
    raw_tab = _raw16(x_table)
    raw_col = _raw16(x_column)
    raw_pred = _raw16(x_predicate)
    zeros = jnp.zeros((_NZR, 16), F32)

    ndst_real = [NOP, NOP, NOP, NPRED, NOP, NTAB, NCOL]
    eis = [ei_scannedby, ei_filters, ei_outputby, ei_connects, ei_calledby,
           ei_sl_table, ei_sl_column]
    edges = []
    for e, nd in zip(eis, ndst_real):
        ne = e.shape[1]
        npad = (-ne) % _CHUNK
        es = jnp.concatenate([e[0], jnp.zeros((npad,), I32)])
        ed = jnp.concatenate([e[1], jnp.full((npad,), nd, I32)])
        edges.append((es.reshape(-1, _JW), ed.reshape(-1, _JW)))

    p16 = {"op": _pad16(W_op, b_op), "tab": _pad16(W_tab, b_tab),
           "col": _pad16(W_col, b_col), "pred": _pad16(W_pred, b_pred)}
    srcs = ["tab", "pred", "col", "col", "op", "tab", "col"]
    a1 = [p16[srcs[i]] @ Wrel1[i] for i in range(7)]
    rel_of_dst = {"op": [0, 1, 2, 4], "pred": [3], "tab": [5], "col": [6]}
    r1 = {}
    cb1 = {}
    for t, rl in rel_of_dst.items():
        wroot_sum = sum(Wroot1[i] for i in rl)
        r1[t] = p16[t] @ wroot_sum
        cb1[t] = sum(brel1[i] for i in rl)[None, :]

    aggs = _sc_agg1([raw_op, raw_tab, raw_col, raw_pred], zeros, edges)

    y2 = {}
    y2["op"], h1_op = _combine1(
        [aggs[0], aggs[1], aggs[2], aggs[4]], raw_op,
        [a1[0], a1[1], a1[2], a1[4]], r1["op"], cb1["op"], Wrel2[4],
        NOP, True)
    y2["tab"], _ = _combine1([aggs[5]], raw_tab, [a1[5]], r1["tab"],
                             cb1["tab"], Wrel2[0], NTAB, False)
    y2["pred"], _ = _combine1([aggs[3]], raw_pred, [a1[3]], r1["pred"],
                              cb1["pred"], Wrel2[1], NPRED, False)
    y2["col"], _ = _combine1([aggs[6]], raw_col, [a1[6]], r1["col"],
                             cb1["col"], Wrel2[2], NCOL, False)

    acc2 = _sc_agg2(y2, [edges[0], edges[1], edges[2], edges[4]], zeros)

    wroot2s = Wroot2[0] + Wroot2[1] + Wroot2[2] + Wroot2[4]
    bias2 = (brel2[0] + brel2[1] + brel2[2] + brel2[4])[None, :]
    batch3d = batch_operator.reshape(NOP // _BLK, 1, _BLK)
    sums, cnts = _head(acc2, h1_op, batch3d, wroot2s, bias2, W_out)
    return sums[:, 0] / jnp.maximum(cnts[:, 0], 1.0) + b_out


# speedup vs baseline: 3.3788x; 1.4122x over previous
import jax
import jax.numpy as jnp
from jax import lax
from jax.experimental import pallas as pl
from jax.experimental.pallas import tpu as pltpu
from jax.experimental.pallas import tpu_sc as plsc

F32 = jnp.float32
I32 = jnp.int32

H = 128
G = 64
NOP, NTAB, NCOL, NPRED = 100000, 20000, 100000, 50000

_JW = 128
_JROWS = 8
_CHUNK = _JW * _JROWS
_NSUB = 16

NOP_P = 100096
NTAB_P = 20096
NPRED_P = 50048
NCOL_P = 100096
_NZR = NOP_P // _NSUB

_mesh = plsc.VectorSubcoreMesh(
    core_axis_name="c", subcore_axis_name="s", num_cores=2, num_subcores=16)


def _edge_chunks(es, ed, raw, acc, idx_s, idx_d, rows3, semg, sems, zeros,
                 sid, col=None):
    nrows = es.shape[0]
    nchunks = nrows // _JROWS
    nk = (nchunks + _NSUB - 1) // _NSUB

    def kbody(k, carry):
        c = k * _NSUB + sid

        @pl.when(c < nchunks)
        def _():
            pltpu.sync_copy(es.at[pl.ds(c * _JROWS, _JROWS), :], idx_s)
            pltpu.sync_copy(ed.at[pl.ds(c * _JROWS, _JROWS), :], idx_d)
            if col is not None:
                for j in range(_JROWS):
                    for l in range(_JW // 16):
                        v = idx_s[j, pl.ds(l * 16, 16)]
                        idx_s[j, pl.ds(l * 16, 16)] = v * 8 + col

            def fire_g(j, carry2):
                pltpu.async_copy(raw.at[idx_s.at[j]], rows3.at[j], semg)
                return carry2

            def drain_g(j, carry2):
                pltpu.make_async_copy(
                    zeros.at[pl.ds(0, _JW)], rows3.at[j], semg).wait()
                return carry2

            def fire_s(j, carry2):
                pltpu.async_copy(rows3.at[j], acc.at[idx_d.at[j]], sems,
                                 add=True)
                return carry2

            def drain_s(j, carry2):
                pltpu.make_async_copy(
                    zeros.at[pl.ds(0, _JW)], rows3.at[j], sems).wait()
                return carry2

            lax.fori_loop(0, _JROWS, fire_g, 0)
            lax.fori_loop(0, _JROWS, drain_g, 0)
            lax.fori_loop(0, _JROWS, fire_s, 0)
            lax.fori_loop(0, _JROWS, drain_s, 0)
        return carry

    lax.fori_loop(0, nk, kbody, 0)


def _sc_agg1_body(raw_op, raw_tab, raw_col, raw_pred, zeros,
                  es0, ed0, es1, ed1, es2, ed2, es3, ed3, es4, ed4,
                  es5, ed5, es6, ed6,
                  agg0, agg1, agg2, agg3, agg4, agg5, agg6,
                  acc, idx_s, idx_d, rows3, semg, sems):
    cid = lax.axis_index("c")
    sid = lax.axis_index("s")
    rels = [
        (es0, ed0, agg0, raw_tab, NOP_P, 0),
        (es2, ed2, agg2, raw_col, NOP_P, 0),
        (es3, ed3, agg3, raw_col, NPRED_P, 0),
        (es1, ed1, agg1, raw_pred, NOP_P, 1),
        (es4, ed4, agg4, raw_op, NOP_P, 1),
        (es5, ed5, agg5, raw_tab, NTAB_P, 1),
        (es6, ed6, agg6, raw_col, NCOL_P, 1),
    ]
    for es, ed, agg, raw, ndst, core in rels:
        @pl.when(cid == core)
        def _(es=es, ed=ed, agg=agg, raw=raw, ndst=ndst):
            nr = ndst // _NSUB
            pltpu.sync_copy(zeros.at[pl.ds(0, nr)],
                            acc.at[pl.ds(sid * nr, nr)])
            plsc.subcore_barrier()
            _edge_chunks(es, ed, raw, acc, idx_s, idx_d, rows3, semg, sems,
                         zeros, sid)
            plsc.subcore_barrier()
            pltpu.sync_copy(acc.at[pl.ds(sid * nr, nr)],
                            agg.at[pl.ds(sid * nr, nr)])
            plsc.subcore_barrier()


def _sc_agg1(raw16s, zeros, edges):
    ndsts = [NOP_P, NOP_P, NOP_P, NPRED_P, NOP_P, NTAB_P, NCOL_P]
    out_type = [jax.ShapeDtypeStruct((n, 16), F32) for n in ndsts]
    flat_edges = [a for pair in edges for a in pair]
    f = pl.kernel(
        _sc_agg1_body,
        out_type=out_type,
        mesh=_mesh,
        scratch_types=[
            pltpu.VMEM_SHARED((NOP_P, 16), F32),
            pltpu.VMEM((_JROWS, _JW), I32),
            pltpu.VMEM((_JROWS, _JW), I32),
            pltpu.VMEM((_JROWS, _JW, 16), F32),
            pltpu.SemaphoreType.DMA,
            pltpu.SemaphoreType.DMA,
        ],
        compiler_params=pltpu.CompilerParams(use_tc_tiling_on_sc=False),
    )
    return f(*raw16s, zeros, *flat_edges)


def _sc_agg2_body(y2_tab, y2_pred, y2_col, y2_op,
                  es0, ed0, es1, ed1, es2, ed2, es4, ed4, zeros, out,
                  acc, idx_s, idx_d, rows3, semg, sems):
    cid = lax.axis_index("c")
    sid = lax.axis_index("s")
    nr = NOP_P // _NSUB
    rel_edges = [(es0, ed0, y2_tab), (es1, ed1, y2_pred),
                 (es2, ed2, y2_col), (es4, ed4, y2_op)]
    for s in range(8):
        @pl.when(cid == s // 4)
        def _(s=s):
            pltpu.sync_copy(zeros.at[pl.ds(0, nr)],
                            acc.at[pl.ds(sid * nr, nr)])
            plsc.subcore_barrier()
            for es, ed, y2 in rel_edges:
                _edge_chunks(es, ed, y2, acc, idx_s, idx_d, rows3, semg,
                             sems, zeros, sid, col=s)
            plsc.subcore_barrier()
            pltpu.sync_copy(acc.at[pl.ds(sid * nr, nr)],
                            out.at[pl.ds(sid * nr, nr), pl.ds(s * 16, 16)])
            plsc.subcore_barrier()


def _sc_agg2(y2, edges, zeros):
    out_type = jax.ShapeDtypeStruct((NOP_P, H), F32)
    args = [y2[t].reshape(-1, 16) for t in ("tab", "pred", "col", "op")]
    for es, ed in edges:
        args.extend((es, ed))
    args.append(zeros)
    f = pl.kernel(
        _sc_agg2_body,
        out_type=out_type,
        mesh=_mesh,
        scratch_types=[
            pltpu.VMEM_SHARED((NOP_P, 16), F32),
            pltpu.VMEM((_JROWS, _JW), I32),
            pltpu.VMEM((_JROWS, _JW), I32),
            pltpu.VMEM((_JROWS, _JW, 16), F32),
            pltpu.SemaphoreType.DMA,
            pltpu.SemaphoreType.DMA,
        ],
        compiler_params=pltpu.CompilerParams(use_tc_tiling_on_sc=False),
    )
    return f(*args)


_BLK = 2000


def _combine1(aggs, raw16, a1s, r1, cb1, wrel2, n, want_h1):
    nb = n // _BLK
    na = len(aggs)

    def body(*refs):
        ins = refs[:na + 1]
        ws = refs[na + 1:2 * na + 1]
        r1_ref, cb1_ref, w2_ref = refs[2 * na + 1:2 * na + 4]
        outs = refs[2 * na + 4:]
        h = jnp.dot(ins[na][...], r1_ref[...],
                    preferred_element_type=F32) + cb1_ref[...]
        for a in range(na):
            h += jnp.dot(ins[a][...], ws[a][...], preferred_element_type=F32)
        h = jnp.maximum(h, 0.0)
        outs[0][...] = jnp.dot(h, w2_ref[...], preferred_element_type=F32)
        if want_h1:
            outs[1][...] = h

    agg_spec = pl.BlockSpec((_BLK, 16), lambda i: (i, 0))
    w_spec = pl.BlockSpec((16, H), lambda i: (0, 0))
    big_spec = pl.BlockSpec((_BLK, H), lambda i: (i, 0))
    out_shapes = [jax.ShapeDtypeStruct((n, H), F32)]
    out_specs = [big_spec]
    if want_h1:
        out_shapes.append(jax.ShapeDtypeStruct((n, H), F32))
        out_specs.append(big_spec)
    res = pl.pallas_call(
        body,
        grid=(nb,),
        in_specs=([agg_spec] * (na + 1) + [w_spec] * (na + 1)
                  + [pl.BlockSpec((1, H), lambda i: (0, 0)),
                     pl.BlockSpec((H, H), lambda i: (0, 0))]),
        out_specs=out_specs,
        out_shape=out_shapes,
    )(*aggs, raw16, *a1s, r1, cb1, wrel2)
    return (res[0], res[1]) if want_h1 else (res[0], None)


def _head(acc2, h1_op, batch3d, wroot2s, bias2, w_out):
    nb = NOP // _BLK

    def body(*refs):
        acc_ref, h1_ref, b_ref, wr_ref, bias_ref, wout_ref = refs[:6]
        sums_ref, cnts_ref = refs[6:8]
        i = pl.program_id(0)
        h2 = jnp.maximum(
            acc_ref[...] + jnp.dot(h1_ref[...], wr_ref[...],
                                   preferred_element_type=F32)
            + bias_ref[...], 0.0)
        z = jnp.dot(h2, wout_ref[...], preferred_element_type=F32)
        b = b_ref[0]
        oh = (lax.broadcasted_iota(I32, (G, _BLK), 0) == b).astype(F32)
        ps = lax.dot_general(oh, z, (((1,), (0,)), ((), ())),
                             preferred_element_type=F32)
        pc = jnp.sum(oh, axis=1, keepdims=True)

        @pl.when(i == 0)
        def _():
            sums_ref[...] = ps
            cnts_ref[...] = pc

        @pl.when(i > 0)
        def _():
            sums_ref[...] += ps
            cnts_ref[...] += pc

    out_spec = pl.BlockSpec((G, 1), lambda i: (0, 0))
    sums, cnts = pl.pallas_call(
        body,
        grid=(nb,),
        in_specs=[pl.BlockSpec((_BLK, H), lambda i: (i, 0)),
                  pl.BlockSpec((_BLK, H), lambda i: (i, 0)),
                  pl.BlockSpec((1, 1, _BLK), lambda i: (i, 0, 0)),
                  pl.BlockSpec((H, H), lambda i: (0, 0)),
                  pl.BlockSpec((1, H), lambda i: (0, 0)),
                  pl.BlockSpec((H, 1), lambda i: (0, 0))],
        out_specs=[out_spec, out_spec],
        out_shape=[jax.ShapeDtypeStruct((G, 1), F32),
                   jax.ShapeDtypeStruct((G, 1), F32)],
    )(acc2, h1_op, batch3d, wroot2s, bias2, w_out)
    return sums, cnts


def _pad16(w, b):
    p = jnp.concatenate([w, b[None, :]], axis=0)
    return jnp.pad(p, ((0, 16 - p.shape[0]), (0, 0)))


def _raw16(x):
    n, d = x.shape
    return jnp.concatenate(
        [x, jnp.ones((n, 1), F32), jnp.zeros((n, 15 - d), F32)], axis=1)


def kernel(x_operator, x_table, x_column, x_predicate, ei_scannedby,
           ei_filters, ei_outputby, ei_connects, ei_calledby, ei_sl_table,
           ei_sl_column, batch_operator, W_op, b_op, W_tab, b_tab, W_col,
           b_col, W_pred, b_pred, Wrel1, brel1, Wroot1, Wrel2, brel2,
           Wroot2, W_out, b_out):
    raw_op = _raw16(x_operator)
    raw_tab = _raw16(x_table)
    raw_col = _raw16(x_column)
    raw_pred = _raw16(x_predicate)
    zeros = jnp.zeros((_NZR, 16), F32)

    ndst_real = [NOP, NOP, NOP, NPRED, NOP, NTAB, NCOL]
    eis = [ei_scannedby, ei_filters, ei_outputby, ei_connects, ei_calledby,
           ei_sl_table, ei_sl_column]
    edges = []
    for e, nd in zip(eis, ndst_real):
        ne = e.shape[1]
        npad = (-ne) % _CHUNK
        es = jnp.concatenate([e[0], jnp.zeros((npad,), I32)])
        ed = jnp.concatenate([e[1], jnp.full((npad,), nd, I32)])
        edges.append((es.reshape(-1, _JW), ed.reshape(-1, _JW)))

    p16 = {"op": _pad16(W_op, b_op), "tab": _pad16(W_tab, b_tab),
           "col": _pad16(W_col, b_col), "pred": _pad16(W_pred, b_pred)}
    srcs = ["tab", "pred", "col", "col", "op", "tab", "col"]
    a1 = [p16[srcs[i]] @ Wrel1[i] for i in range(7)]
    rel_of_dst = {"op": [0, 1, 2, 4], "pred": [3], "tab": [5], "col": [6]}
    r1 = {}
    cb1 = {}
    for t, rl in rel_of_dst.items():
        wroot_sum = sum(Wroot1[i] for i in rl)
        r1[t] = p16[t] @ wroot_sum
        cb1[t] = sum(brel1[i] for i in rl)[None, :]

    aggs = _sc_agg1([raw_op, raw_tab, raw_col, raw_pred], zeros, edges)

    y2 = {}
    y2["op"], h1_op = _combine1(
        [aggs[0], aggs[1], aggs[2], aggs[4]], raw_op,
        [a1[0], a1[1], a1[2], a1[4]], r1["op"], cb1["op"], Wrel2[4],
        NOP, True)
    y2["tab"], _ = _combine1([aggs[5]], raw_tab, [a1[5]], r1["tab"],
                             cb1["tab"], Wrel2[0], NTAB, False)
    y2["pred"], _ = _combine1([aggs[3]], raw_pred, [a1[3]], r1["pred"],
                              cb1["pred"], Wrel2[1], NPRED, False)
    y2["col"], _ = _combine1([aggs[6]], raw_col, [a1[6]], r1["col"],
                             cb1["col"], Wrel2[2], NCOL, False)

    acc2 = _sc_agg2(y2, [edges[0], edges[1], edges[2], edges[4]], zeros)

    wroot2s = Wroot2[0] + Wroot2[1] + Wroot2[2] + Wroot2[4]
    bias2 = (brel2[0] + brel2[1] + brel2[2] + brel2[4])[None, :]
    batch3d = batch_operator.reshape(NOP // _BLK, 1, _BLK)
    sums, cnts = _head(acc2, h1_op, batch3d, wroot2s, bias2, W_out)
    return sums[:, 0] / jnp.maximum(cnts[:, 0], 1.0) + b_out


# speedup vs baseline: 3.5525x; 1.0514x over previous
import jax
import jax.numpy as jnp
from jax import lax
from jax.experimental import pallas as pl
from jax.experimental.pallas import tpu as pltpu
from jax.experimental.pallas import tpu_sc as plsc

F32 = jnp.float32
I32 = jnp.int32

H = 128
G = 64
NOP, NTAB, NCOL, NPRED = 100000, 20000, 100000, 50000

_JW = 128
_JROWS = 4
_CHUNK = _JW * _JROWS
_NSUB = 16

NOP_P = 100096
NTAB_P = 20096
NPRED_P = 50048
NCOL_P = 100096
_NZR = NOP_P // _NSUB

_mesh = plsc.VectorSubcoreMesh(
    core_axis_name="c", subcore_axis_name="s", num_cores=2, num_subcores=16)


def _edge_chunks(es, ed, raw, acc, idx_s, idx_d, rows3, semg, sems, zeros,
                 sid, col=None):
    nrows = es.shape[0]
    nchunks = nrows // _JROWS
    my_n = (nchunks - sid + _NSUB - 1) // _NSUB

    def load_idx(k, p):
        c = k * _NSUB + sid
        pltpu.sync_copy(es.at[pl.ds(c * _JROWS, _JROWS), :], idx_s.at[p])
        pltpu.sync_copy(ed.at[pl.ds(c * _JROWS, _JROWS), :], idx_d.at[p])
        if col is not None:
            for j in range(_JROWS):
                for l in range(_JW // 16):
                    v = idx_s[p, j, pl.ds(l * 16, 16)]
                    idx_s[p, j, pl.ds(l * 16, 16)] = v * 8 + col

    def fire_g(p):
        def f(j, carry2):
            pltpu.async_copy(raw.at[idx_s.at[p].at[j]], rows3.at[p].at[j],
                             semg)
            return carry2
        lax.fori_loop(0, _JROWS, f, 0)

    def fire_s(p):
        def f(j, carry2):
            pltpu.async_copy(rows3.at[p].at[j], acc.at[idx_d.at[p].at[j]],
                             sems, add=True)
            return carry2
        lax.fori_loop(0, _JROWS, f, 0)

    def drain(sem):
        def f(j, carry2):
            pltpu.make_async_copy(
                zeros.at[pl.ds(0, _JW)], rows3.at[0].at[j], sem).wait()
            return carry2
        lax.fori_loop(0, _JROWS, f, 0)

    @pl.when(my_n > 0)
    def _():
        load_idx(0, 0)
        fire_g(0)

        def kbody(k, carry):
            p = k % 2

            @pl.when(k > 0)
            def _():
                drain(sems)

            @pl.when(k + 1 < my_n)
            def _():
                load_idx(k + 1, 1 - p)
            drain(semg)

            @pl.when(k + 1 < my_n)
            def _():
                fire_g(1 - p)
            fire_s(p)
            return carry

        lax.fori_loop(0, my_n, kbody, 0)
        drain(sems)


def _sc_agg1_body(raw_op, raw_tab, raw_col, raw_pred, zeros,
                  es0, ed0, es1, ed1, es2, ed2, es3, ed3, es4, ed4,
                  es5, ed5, es6, ed6,
                  agg0, agg1, agg2, agg3, agg4, agg5, agg6,
                  acc, idx_s, idx_d, rows3, semg, sems):
    cid = lax.axis_index("c")
    sid = lax.axis_index("s")
    rels = [
        (es0, ed0, agg0, raw_tab, NOP_P, 0),
        (es2, ed2, agg2, raw_col, NOP_P, 0),
        (es3, ed3, agg3, raw_col, NPRED_P, 0),
        (es1, ed1, agg1, raw_pred, NOP_P, 1),
        (es4, ed4, agg4, raw_op, NOP_P, 1),
        (es5, ed5, agg5, raw_tab, NTAB_P, 1),
        (es6, ed6, agg6, raw_col, NCOL_P, 1),
    ]
    for es, ed, agg, raw, ndst, core in rels:
        @pl.when(cid == core)
        def _(es=es, ed=ed, agg=agg, raw=raw, ndst=ndst):
            nr = ndst // _NSUB
            pltpu.sync_copy(zeros.at[pl.ds(0, nr)],
                            acc.at[pl.ds(sid * nr, nr)])
            plsc.subcore_barrier()
            _edge_chunks(es, ed, raw, acc, idx_s, idx_d, rows3, semg, sems,
                         zeros, sid)
            plsc.subcore_barrier()
            pltpu.sync_copy(acc.at[pl.ds(sid * nr, nr)],
                            agg.at[pl.ds(sid * nr, nr)])
            plsc.subcore_barrier()


def _sc_agg1(raw16s, zeros, edges):
    ndsts = [NOP_P, NOP_P, NOP_P, NPRED_P, NOP_P, NTAB_P, NCOL_P]
    out_type = [jax.ShapeDtypeStruct((n, 16), F32) for n in ndsts]
    flat_edges = [a for pair in edges for a in pair]
    f = pl.kernel(
        _sc_agg1_body,
        out_type=out_type,
        mesh=_mesh,
        scratch_types=[
            pltpu.VMEM_SHARED((NOP_P, 16), F32),
            pltpu.VMEM((2, _JROWS, _JW), I32),
            pltpu.VMEM((2, _JROWS, _JW), I32),
            pltpu.VMEM((2, _JROWS, _JW, 16), F32),
            pltpu.SemaphoreType.DMA,
            pltpu.SemaphoreType.DMA,
        ],
        compiler_params=pltpu.CompilerParams(use_tc_tiling_on_sc=False),
    )
    return f(*raw16s, zeros, *flat_edges)


def _sc_agg2_body(y2_tab, y2_pred, y2_col, y2_op,
                  es0, ed0, es1, ed1, es2, ed2, es4, ed4, zeros, out,
                  acc, idx_s, idx_d, rows3, semg, sems):
    cid = lax.axis_index("c")
    sid = lax.axis_index("s")
    nr = NOP_P // _NSUB
    rel_edges = [(es0, ed0, y2_tab), (es1, ed1, y2_pred),
                 (es2, ed2, y2_col), (es4, ed4, y2_op)]
    for s in range(8):
        @pl.when(cid == s // 4)
        def _(s=s):
            pltpu.sync_copy(zeros.at[pl.ds(0, nr)],
                            acc.at[pl.ds(sid * nr, nr)])
            plsc.subcore_barrier()
            for es, ed, y2 in rel_edges:
                _edge_chunks(es, ed, y2, acc, idx_s, idx_d, rows3, semg,
                             sems, zeros, sid, col=s)
            plsc.subcore_barrier()
            pltpu.sync_copy(acc.at[pl.ds(sid * nr, nr)],
                            out.at[pl.ds(sid * nr, nr), pl.ds(s * 16, 16)])
            plsc.subcore_barrier()


def _sc_agg2(y2, edges, zeros):
    out_type = jax.ShapeDtypeStruct((NOP_P, H), F32)
    args = [y2[t].reshape(-1, 16) for t in ("tab", "pred", "col", "op")]
    for es, ed in edges:
        args.extend((es, ed))
    args.append(zeros)
    f = pl.kernel(
        _sc_agg2_body,
        out_type=out_type,
        mesh=_mesh,
        scratch_types=[
            pltpu.VMEM_SHARED((NOP_P, 16), F32),
            pltpu.VMEM((2, _JROWS, _JW), I32),
            pltpu.VMEM((2, _JROWS, _JW), I32),
            pltpu.VMEM((2, _JROWS, _JW, 16), F32),
            pltpu.SemaphoreType.DMA,
            pltpu.SemaphoreType.DMA,
        ],
        compiler_params=pltpu.CompilerParams(use_tc_tiling_on_sc=False),
    )
    return f(*args)


_BLK = 2000


def _combine1(aggs, raw16, a1s, r1, cb1, wrel2, n, want_h1):
    nb = n // _BLK
    na = len(aggs)

    def body(*refs):
        ins = refs[:na + 1]
        ws = refs[na + 1:2 * na + 1]
        r1_ref, cb1_ref, w2_ref = refs[2 * na + 1:2 * na + 4]
        outs = refs[2 * na + 4:]
        h = jnp.dot(ins[na][...], r1_ref[...],
                    preferred_element_type=F32) + cb1_ref[...]
        for a in range(na):
            h += jnp.dot(ins[a][...], ws[a][...], preferred_element_type=F32)
        h = jnp.maximum(h, 0.0)
        outs[0][...] = jnp.dot(h, w2_ref[...], preferred_element_type=F32)
        if want_h1:
            outs[1][...] = h

    agg_spec = pl.BlockSpec((_BLK, 16), lambda i: (i, 0))
    w_spec = pl.BlockSpec((16, H), lambda i: (0, 0))
    big_spec = pl.BlockSpec((_BLK, H), lambda i: (i, 0))
    out_shapes = [jax.ShapeDtypeStruct((n, H), F32)]
    out_specs = [big_spec]
    if want_h1:
        out_shapes.append(jax.ShapeDtypeStruct((n, H), F32))
        out_specs.append(big_spec)
    res = pl.pallas_call(
        body,
        grid=(nb,),
        in_specs=([agg_spec] * (na + 1) + [w_spec] * (na + 1)
                  + [pl.BlockSpec((1, H), lambda i: (0, 0)),
                     pl.BlockSpec((H, H), lambda i: (0, 0))]),
        out_specs=out_specs,
        out_shape=out_shapes,
    )(*aggs, raw16, *a1s, r1, cb1, wrel2)
    return (res[0], res[1]) if want_h1 else (res[0], None)


def _head(acc2, h1_op, batch3d, wroot2s, bias2, w_out):
    nb = NOP // _BLK

    def body(*refs):
        acc_ref, h1_ref, b_ref, wr_ref, bias_ref, wout_ref = refs[:6]
        sums_ref, cnts_ref = refs[6:8]
        i = pl.program_id(0)
        h2 = jnp.maximum(
            acc_ref[...] + jnp.dot(h1_ref[...], wr_ref[...],
                                   preferred_element_type=F32)
            + bias_ref[...], 0.0)
        z = jnp.dot(h2, wout_ref[...], preferred_element_type=F32)
        b = b_ref[0]
        oh = (lax.broadcasted_iota(I32, (G, _BLK), 0) == b).astype(F32)
        ps = lax.dot_general(oh, z, (((1,), (0,)), ((), ())),
                             preferred_element_type=F32)
        pc = jnp.sum(oh, axis=1, keepdims=True)

        @pl.when(i == 0)
        def _():
            sums_ref[...] = ps
            cnts_ref[...] = pc

        @pl.when(i > 0)
        def _():
            sums_ref[...] += ps
            cnts_ref[...] += pc

    out_spec = pl.BlockSpec((G, 1), lambda i: (0, 0))
    sums, cnts = pl.pallas_call(
        body,
        grid=(nb,),
        in_specs=[pl.BlockSpec((_BLK, H), lambda i: (i, 0)),
                  pl.BlockSpec((_BLK, H), lambda i: (i, 0)),
                  pl.BlockSpec((1, 1, _BLK), lambda i: (i, 0, 0)),
                  pl.BlockSpec((H, H), lambda i: (0, 0)),
                  pl.BlockSpec((1, H), lambda i: (0, 0)),
                  pl.BlockSpec((H, 1), lambda i: (0, 0))],
        out_specs=[out_spec, out_spec],
        out_shape=[jax.ShapeDtypeStruct((G, 1), F32),
                   jax.ShapeDtypeStruct((G, 1), F32)],
    )(acc2, h1_op, batch3d, wroot2s, bias2, w_out)
    return sums, cnts


def _pad16(w, b):
    p = jnp.concatenate([w, b[None, :]], axis=0)
    return jnp.pad(p, ((0, 16 - p.shape[0]), (0, 0)))


def _raw16(x):
    n, d = x.shape
    return jnp.concatenate(
        [x, jnp.ones((n, 1), F32), jnp.zeros((n, 15 - d), F32)], axis=1)


def kernel(x_operator, x_table, x_column, x_predicate, ei_scannedby,
           ei_filters, ei_outputby, ei_connects, ei_calledby, ei_sl_table,
           ei_sl_column, batch_operator, W_op, b_op, W_tab, b_tab, W_col,
           b_col, W_pred, b_pred, Wrel1, brel1, Wroot1, Wrel2, brel2,
           Wroot2, W_out, b_out):
    raw_op = _raw16(x_operator)
    raw_tab = _raw16(x_table)
    raw_col = _raw16(x_column)
    raw_pred = _raw16(x_predicate)
    zeros = jnp.zeros((_NZR, 16), F32)

    ndst_real = [NOP, NOP, NOP, NPRED, NOP, NTAB, NCOL]
    eis = [ei_scannedby, ei_filters, ei_outputby, ei_connects, ei_calledby,
           ei_sl_table, ei_sl_column]
    edges = []
    for e, nd in zip(eis, ndst_real):
        ne = e.shape[1]
        npad = (-ne) % _CHUNK
        es = jnp.concatenate([e[0], jnp.zeros((npad,), I32)])
        ed = jnp.concatenate([e[1], jnp.full((npad,), nd, I32)])
        edges.append((es.reshape(-1, _JW), ed.reshape(-1, _JW)))

    p16 = {"op": _pad16(W_op, b_op), "tab": _pad16(W_tab, b_tab),
           "col": _pad16(W_col, b_col), "pred": _pad16(W_pred, b_pred)}
    srcs = ["tab", "pred", "col", "col", "op", "tab", "col"]
    a1 = [p16[srcs[i]] @ Wrel1[i] for i in range(7)]
    rel_of_dst = {"op": [0, 1, 2, 4], "pred": [3], "tab": [5], "col": [6]}
    r1 = {}
    cb1 = {}
    for t, rl in rel_of_dst.items():
        wroot_sum = sum(Wroot1[i] for i in rl)
        r1[t] = p16[t] @ wroot_sum
        cb1[t] = sum(brel1[i] for i in rl)[None, :]

    aggs = _sc_agg1([raw_op, raw_tab, raw_col, raw_pred], zeros, edges)

    y2 = {}
    y2["op"], h1_op = _combine1(
        [aggs[0], aggs[1], aggs[2], aggs[4]], raw_op,
        [a1[0], a1[1], a1[2], a1[4]], r1["op"], cb1["op"], Wrel2[4],
        NOP, True)
    y2["tab"], _ = _combine1([aggs[5]], raw_tab, [a1[5]], r1["tab"],
                             cb1["tab"], Wrel2[0], NTAB, False)
    y2["pred"], _ = _combine1([aggs[3]], raw_pred, [a1[3]], r1["pred"],
                              cb1["pred"], Wrel2[1], NPRED, False)
    y2["col"], _ = _combine1([aggs[6]], raw_col, [a1[6]], r1["col"],
                             cb1["col"], Wrel2[2], NCOL, False)

    acc2 = _sc_agg2(y2, [edges[0], edges[1], edges[2], edges[4]], zeros)

    wroot2s = Wroot2[0] + Wroot2[1] + Wroot2[2] + Wroot2[4]
    bias2 = (brel2[0] + brel2[1] + brel2[2] + brel2[4])[None, :]
    batch3d = batch_operator.reshape(NOP // _BLK, 1, _BLK)
    sums, cnts = _head(acc2, h1_op, batch3d, wroot2s, bias2, W_out)
    return sums[:, 0] / jnp.maximum(cnts[:, 0], 1.0) + b_out
